# Initial kernel scaffold; baseline (speedup 1.0000x reference)
import jax
import jax.numpy as jnp
from jax.experimental import pallas as pl


def kernel(text, table, W1, b1, W2, b2, W3, b3):
    raise NotImplementedError("write your pallas kernel here")


# speedup vs baseline: 7.7633x; 7.7633x over previous
import functools

import jax
import jax.numpy as jnp
from jax import lax
from jax.experimental import pallas as pl
from jax.experimental.pallas import tpu as pltpu
from jax.experimental.pallas import tpu_sc as plsc

_EMB = 128
_LANES = 16
_VPR = _EMB // _LANES

_NC = 2
_NS = 16
_NW = _NC * _NS

_GATHER = 80


def _pool_body(nb, nchunk, L, text_hbm, table_hbm, out_hbm, idx_v, rows_v,
               stage_v, sem):
    rows = nb * L
    c = lax.axis_index("c")
    s = lax.axis_index("s")
    wid = s * _NC + c
    base_b = wid * (nb * nchunk)

    @pl.loop(0, nchunk)
    def _chunk(ci):
        b0 = base_b + ci * nb
        pltpu.sync_copy(text_hbm.at[pl.ds(b0 * L, rows)], idx_v)
        for g in range(rows // _GATHER):
            pltpu.async_copy(
                table_hbm.at[idx_v.at[pl.ds(g * _GATHER, _GATHER)]],
                rows_v.at[pl.ds(g * _GATHER, _GATHER)], sem)
        for g in range(rows // _GATHER):
            pltpu.make_async_copy(
                table_hbm.at[idx_v.at[pl.ds(g * _GATHER, _GATHER)]],
                rows_v.at[pl.ds(g * _GATHER, _GATHER)], sem).wait()
        for b in range(nb):
            def lbody(l, accs, b=b):
                r = b * L + l
                return tuple(accs[j] + rows_v[r, pl.ds(j * _LANES, _LANES)]
                             for j in range(_VPR))
            accs = lax.fori_loop(
                0, L, lbody,
                tuple(jnp.zeros((_LANES,), jnp.float32) for _ in range(_VPR)))
            for j in range(_VPR):
                stage_v[b, pl.ds(j * _LANES, _LANES)] = accs[j]
        pltpu.sync_copy(stage_v, out_hbm.at[pl.ds(b0, nb)])


def _make_pool(B, L):
    nb = 8
    bpw = B // _NW
    nchunk = bpw // nb
    rows = nb * L
    mesh = plsc.VectorSubcoreMesh(core_axis_name="c", subcore_axis_name="s")
    return pl.kernel(
        functools.partial(_pool_body, nb, nchunk, L),
        out_type=jax.ShapeDtypeStruct((B, _EMB), jnp.float32),
        mesh=mesh,
        scratch_types=[
            pltpu.VMEM((rows,), jnp.int32),
            pltpu.VMEM((rows, _EMB), jnp.float32),
            pltpu.VMEM((nb, _EMB), jnp.float32),
            pltpu.SemaphoreType.DMA,
        ],
    )


def _mlp_body(text_ref, ps_ref, W1_ref, b1_ref, W2_ref, b2_ref, W3_ref,
              b3_ref, o_ref):
    mask = (text_ref[...] != 0).astype(jnp.float32)
    cnt = jnp.maximum(jnp.sum(mask, axis=1, keepdims=True), 1.0)
    pooled = ps_ref[...] / cnt
    dn = (((1,), (1,)), ((), ()))
    h = lax.dot_general(pooled, W1_ref[...], dn,
                        preferred_element_type=jnp.float32) + b1_ref[...]
    h = jnp.maximum(h, 0.0)
    h = lax.dot_general(h, W2_ref[...], dn,
                        preferred_element_type=jnp.float32) + b2_ref[...]
    h = jnp.maximum(h, 0.0)
    o_ref[...] = lax.dot_general(h, W3_ref[...], dn,
                                 preferred_element_type=jnp.float32) + b3_ref[...]


def _mlp(text, pooled_sum, W1, b1, W2, b2, W3, b3):
    B, L = text.shape
    H1 = W1.shape[0]
    H2 = W2.shape[0]
    OUT = W3.shape[0]
    BM = 512
    grid = (B // BM,)
    full = lambda i: (0, 0)
    return pl.pallas_call(
        _mlp_body,
        grid=grid,
        in_specs=[
            pl.BlockSpec((BM, L), lambda i: (i, 0)),
            pl.BlockSpec((BM, _EMB), lambda i: (i, 0)),
            pl.BlockSpec((H1, _EMB), full),
            pl.BlockSpec((1, H1), full),
            pl.BlockSpec((H2, H1), full),
            pl.BlockSpec((1, H2), full),
            pl.BlockSpec((OUT, H2), full),
            pl.BlockSpec((1, OUT), full),
        ],
        out_specs=pl.BlockSpec((BM, OUT), lambda i: (i, 0)),
        out_shape=jax.ShapeDtypeStruct((B, OUT), jnp.float32),
    )(text, pooled_sum, W1, b1.reshape(1, -1), W2, b2.reshape(1, -1), W3,
      b3.reshape(1, -1))


def kernel(text, table, W1, b1, W2, b2, W3, b3):
    B, L = text.shape
    text_flat = text.reshape(-1).astype(jnp.int32)
    pooled_sum = _make_pool(B, L)(text_flat, table)
    return _mlp(text, pooled_sum, W1, b1, W2, b2, W3, b3)


# speedup vs baseline: 11.0709x; 1.4260x over previous
import functools

import jax
import jax.numpy as jnp
from jax import lax
from jax.experimental import pallas as pl
from jax.experimental.pallas import tpu as pltpu
from jax.experimental.pallas import tpu_sc as plsc

_EMB = 128
_LANES = 16
_VPR = _EMB // _LANES

_NC = 2
_NS = 16
_NW = _NC * _NS

_GATHER = 80


_UNROLL = 5


def _pool_body(nb, nchunk, L, text_hbm, table_hbm, out_hbm, idx_v, rows0,
               rows1, stage_v, sem0, sem1):
    rows = nb * L
    c = lax.axis_index("c")
    s = lax.axis_index("s")
    wid = s * _NC + c
    base_b = wid * (nb * nchunk)
    pltpu.sync_copy(text_hbm.at[pl.ds(base_b * L, nb * nchunk * L)], idx_v)

    bufs = (rows0, rows1)
    sems = (sem0, sem1)

    def fire(ci, k):
        for g in range(rows // _GATHER):
            pltpu.async_copy(
                table_hbm.at[idx_v.at[pl.ds(ci * rows + g * _GATHER, _GATHER)]],
                bufs[k].at[pl.ds(g * _GATHER, _GATHER)], sems[k])

    def drain(ci, k):
        for g in range(rows // _GATHER):
            pltpu.make_async_copy(
                table_hbm.at[idx_v.at[pl.ds(ci * rows + g * _GATHER, _GATHER)]],
                bufs[k].at[pl.ds(g * _GATHER, _GATHER)], sems[k]).wait()

    def consume(ci, k):
        b0 = base_b + ci * nb
        for b in range(nb):
            def lbody(t, accs, b=b, k=k):
                out = list(accs)
                for u in range(_UNROLL):
                    r = b * L + t * _UNROLL + u
                    for j in range(_VPR):
                        out[j] = out[j] + bufs[k][r, pl.ds(j * _LANES, _LANES)]
                return tuple(out)
            accs = lax.fori_loop(
                0, L // _UNROLL, lbody,
                tuple(jnp.zeros((_LANES,), jnp.float32) for _ in range(_VPR)))
            for j in range(_VPR):
                stage_v[b, pl.ds(j * _LANES, _LANES)] = accs[j]
        pltpu.sync_copy(stage_v, out_hbm.at[pl.ds(b0, nb)])

    fire(0, 0)

    @pl.loop(0, nchunk, step=2)
    def _pair(ci):
        fire(ci + 1, 1)
        drain(ci, 0)
        consume(ci, 0)

        @pl.when(ci + 2 < nchunk)
        def _():
            fire(ci + 2, 0)

        drain(ci + 1, 1)
        consume(ci + 1, 1)


def _make_pool(B, L):
    nb = 8
    bpw = B // _NW
    nchunk = bpw // nb
    rows = nb * L
    mesh = plsc.VectorSubcoreMesh(core_axis_name="c", subcore_axis_name="s")
    return pl.kernel(
        functools.partial(_pool_body, nb, nchunk, L),
        out_type=jax.ShapeDtypeStruct((B, _EMB), jnp.float32),
        mesh=mesh,
        scratch_types=[
            pltpu.VMEM((bpw * L,), jnp.int32),
            pltpu.VMEM((rows, _EMB), jnp.float32),
            pltpu.VMEM((rows, _EMB), jnp.float32),
            pltpu.VMEM((nb, _EMB), jnp.float32),
            pltpu.SemaphoreType.DMA,
            pltpu.SemaphoreType.DMA,
        ],
    )


def _mlp_body(text_ref, ps_ref, W1_ref, b1_ref, W2_ref, b2_ref, W3_ref,
              b3_ref, o_ref):
    mask = (text_ref[...] != 0).astype(jnp.float32)
    cnt = jnp.maximum(jnp.sum(mask, axis=1, keepdims=True), 1.0)
    pooled = ps_ref[...] / cnt
    dn = (((1,), (1,)), ((), ()))
    h = lax.dot_general(pooled, W1_ref[...], dn,
                        preferred_element_type=jnp.float32) + b1_ref[...]
    h = jnp.maximum(h, 0.0)
    h = lax.dot_general(h, W2_ref[...], dn,
                        preferred_element_type=jnp.float32) + b2_ref[...]
    h = jnp.maximum(h, 0.0)
    o_ref[...] = lax.dot_general(h, W3_ref[...], dn,
                                 preferred_element_type=jnp.float32) + b3_ref[...]


def _mlp(text, pooled_sum, W1, b1, W2, b2, W3, b3):
    B, L = text.shape
    H1 = W1.shape[0]
    H2 = W2.shape[0]
    OUT = W3.shape[0]
    BM = 512
    grid = (B // BM,)
    full = lambda i: (0, 0)
    return pl.pallas_call(
        _mlp_body,
        grid=grid,
        in_specs=[
            pl.BlockSpec((BM, L), lambda i: (i, 0)),
            pl.BlockSpec((BM, _EMB), lambda i: (i, 0)),
            pl.BlockSpec((H1, _EMB), full),
            pl.BlockSpec((1, H1), full),
            pl.BlockSpec((H2, H1), full),
            pl.BlockSpec((1, H2), full),
            pl.BlockSpec((OUT, H2), full),
            pl.BlockSpec((1, OUT), full),
        ],
        out_specs=pl.BlockSpec((BM, OUT), lambda i: (i, 0)),
        out_shape=jax.ShapeDtypeStruct((B, OUT), jnp.float32),
    )(text, pooled_sum, W1, b1.reshape(1, -1), W2, b2.reshape(1, -1), W3,
      b3.reshape(1, -1))


def kernel(text, table, W1, b1, W2, b2, W3, b3):
    B, L = text.shape
    text_flat = text.reshape(-1).astype(jnp.int32)
    pooled_sum = _make_pool(B, L)(text_flat, table)
    return _mlp(text, pooled_sum, W1, b1, W2, b2, W3, b3)


# speedup vs baseline: 11.0796x; 1.0008x over previous
import functools

import jax
import jax.numpy as jnp
from jax import lax
from jax.experimental import pallas as pl
from jax.experimental.pallas import tpu as pltpu
from jax.experimental.pallas import tpu_sc as plsc

_EMB = 128
_LANES = 16
_VPR = _EMB // _LANES

_NC = 2
_NS = 16
_NW = _NC * _NS

_GATHER = 80


_UNROLL = 5


def _pool_body(nb, nchunk, L, text_hbm, table_hbm, out_hbm, idx_v, rows0,
               rows1, stage_v, sem0, sem1):
    rows = nb * L
    c = lax.axis_index("c")
    s = lax.axis_index("s")
    wid = s * _NC + c
    base_b = wid * (nb * nchunk)
    pltpu.sync_copy(text_hbm.at[pl.ds(base_b * L, nb * nchunk * L)], idx_v)

    bufs = (rows0, rows1)
    sems = (sem0, sem1)

    def fire(ci, k):
        for g in range(rows // _GATHER):
            pltpu.async_copy(
                table_hbm.at[idx_v.at[pl.ds(ci * rows + g * _GATHER, _GATHER)]],
                bufs[k].at[pl.ds(g * _GATHER, _GATHER)], sems[k])

    def drain(ci, k):
        for g in range(rows // _GATHER):
            pltpu.make_async_copy(
                table_hbm.at[idx_v.at[pl.ds(ci * rows + g * _GATHER, _GATHER)]],
                bufs[k].at[pl.ds(g * _GATHER, _GATHER)], sems[k]).wait()

    def consume(ci, k):
        b0 = base_b + ci * nb
        for b in range(nb):
            def lbody(t, accs, b=b, k=k):
                out = list(accs)
                for u in range(_UNROLL):
                    r = b * L + t * _UNROLL + u
                    for j in range(_VPR):
                        out[j] = out[j] + bufs[k][r, pl.ds(j * _LANES, _LANES)]
                return tuple(out)
            accs = lax.fori_loop(
                0, L // _UNROLL, lbody,
                tuple(jnp.zeros((_LANES,), jnp.float32) for _ in range(_VPR)))
            for j in range(_VPR):
                stage_v[b, pl.ds(j * _LANES, _LANES)] = accs[j]
        pltpu.sync_copy(stage_v, out_hbm.at[pl.ds(b0, nb)])

    fire(0, 0)

    @pl.loop(0, nchunk, step=2)
    def _pair(ci):
        fire(ci + 1, 1)
        drain(ci, 0)
        consume(ci, 0)

        @pl.when(ci + 2 < nchunk)
        def _():
            fire(ci + 2, 0)

        drain(ci + 1, 1)
        consume(ci + 1, 1)


def _make_pool(B, L):
    nb = 8
    bpw = B // _NW
    nchunk = bpw // nb
    rows = nb * L
    mesh = plsc.VectorSubcoreMesh(core_axis_name="c", subcore_axis_name="s")
    return pl.kernel(
        functools.partial(_pool_body, nb, nchunk, L),
        out_type=jax.ShapeDtypeStruct((B, _EMB), jnp.float32),
        mesh=mesh,
        scratch_types=[
            pltpu.VMEM((bpw * L,), jnp.int32),
            pltpu.VMEM((rows, _EMB), jnp.float32),
            pltpu.VMEM((rows, _EMB), jnp.float32),
            pltpu.VMEM((nb, _EMB), jnp.float32),
            pltpu.SemaphoreType.DMA,
            pltpu.SemaphoreType.DMA,
        ],
    )


def _mlp_body(text_ref, ps_ref, W1_ref, b1_ref, W2_ref, b2_ref, W3_ref,
              b3_ref, o_ref):
    mask = (text_ref[...] != 0).astype(jnp.float32)
    cnt = jnp.maximum(jnp.sum(mask, axis=1, keepdims=True), 1.0)
    pooled = ps_ref[...] / cnt
    dn = (((1,), (1,)), ((), ()))
    h = lax.dot_general(pooled.astype(jnp.bfloat16), W1_ref[...], dn,
                        preferred_element_type=jnp.float32) + b1_ref[...]
    h = jnp.maximum(h, 0.0)
    h = lax.dot_general(h.astype(jnp.bfloat16), W2_ref[...], dn,
                        preferred_element_type=jnp.float32) + b2_ref[...]
    h = jnp.maximum(h, 0.0)
    o_ref[...] = lax.dot_general(h.astype(jnp.bfloat16), W3_ref[...], dn,
                                 preferred_element_type=jnp.float32) + b3_ref[...]


def _mlp(text, pooled_sum, W1, b1, W2, b2, W3, b3):
    B, L = text.shape
    H1 = W1.shape[0]
    H2 = W2.shape[0]
    OUT = W3.shape[0]
    BM = 512
    grid = (B // BM,)
    full = lambda i: (0, 0)
    return pl.pallas_call(
        _mlp_body,
        grid=grid,
        in_specs=[
            pl.BlockSpec((BM, L), lambda i: (i, 0)),
            pl.BlockSpec((BM, _EMB), lambda i: (i, 0)),
            pl.BlockSpec((H1, _EMB), full),
            pl.BlockSpec((1, H1), full),
            pl.BlockSpec((H2, H1), full),
            pl.BlockSpec((1, H2), full),
            pl.BlockSpec((OUT, H2), full),
            pl.BlockSpec((1, OUT), full),
        ],
        out_specs=pl.BlockSpec((BM, OUT), lambda i: (i, 0)),
        out_shape=jax.ShapeDtypeStruct((B, OUT), jnp.float32),
    )(text, pooled_sum, W1.astype(jnp.bfloat16), b1.reshape(1, -1),
      W2.astype(jnp.bfloat16), b2.reshape(1, -1),
      W3.astype(jnp.bfloat16), b3.reshape(1, -1))


def kernel(text, table, W1, b1, W2, b2, W3, b3):
    B, L = text.shape
    text_flat = text.reshape(-1).astype(jnp.int32)
    pooled_sum = _make_pool(B, L)(text_flat, table)
    return _mlp(text, pooled_sum, W1, b1, W2, b2, W3, b3)
